# baseline (speedup 1.0000x reference)
import math
import functools

import jax
import jax.numpy as jnp
import numpy as np
from jax import lax
from jax.experimental import pallas as pl
from jax.experimental.pallas import tpu as pltpu
from jax.experimental.pallas import tpu_sc as plsc

BSZ = 4
SEQ = 8192
DIM = 1024
NUM_TOKENS = BSZ * SEQ
NW = 32
CHUNK = NUM_TOKENS // NW
CPR = SEQ // CHUNK
G = 16
NG = CHUNK // G
NBUF = 6
LAG = 3
LANES = 16

_HALF = DIM // 2
_EMB_SCALE = math.log(10000.0) / (_HALF - 1)


def _build_table():
    freqs = np.exp(np.arange(_HALF, dtype=np.float32) * -_EMB_SCALE)
    ang = np.arange(SEQ + 1, dtype=np.float32)[:, None] * freqs[None, :]
    tab = np.concatenate([np.sin(ang), np.cos(ang)], axis=1).astype(np.float32)
    tab[0, :] = 0.0
    return tab


_TABLE = _build_table()


def _sc_body(table_hbm, mask_hbm, out_hbm, row_v, pos_v, *bufs_and_sems):
    rows = bufs_and_sems[:NBUF]
    sg = bufs_and_sems[NBUF:2 * NBUF]
    so = bufs_and_sems[2 * NBUF:3 * NBUF]

    wid = lax.axis_index("s") * 2 + lax.axis_index("c")
    b = wid // CPR
    c = wid % CPR
    row_base = b * SEQ
    cbase = c * CHUNK
    out_base = row_base + cbase

    pltpu.sync_copy(mask_hbm.at[pl.ds(row_base, SEQ)], row_v)

    def _ofs(i, acc):
        return acc + jnp.sum(row_v[pl.ds(i * LANES, LANES)])

    offset = lax.fori_loop(0, c * (CHUNK // LANES), _ofs, jnp.int32(0))

    carry = offset
    for i in range(CHUNK // LANES):
        v = row_v[pl.ds(cbase + i * LANES, LANES)]
        cum = plsc.cumsum(v) + carry
        carry = carry + jnp.sum(v)
        pos_v[i, :] = cum * v

    def _gather(s, buf):
        pltpu.async_copy(table_hbm.at[pos_v.at[s]], rows[buf], sg[buf])

    def _gather_wait(s, buf):
        pltpu.make_async_copy(table_hbm.at[pos_v.at[s]], rows[buf], sg[buf]).wait()

    def _out(s, buf):
        pltpu.async_copy(rows[buf], out_hbm.at[pl.ds(out_base + s * G, G)], so[buf])

    def _out_wait(buf):
        pltpu.make_async_copy(rows[buf], out_hbm.at[pl.ds(out_base, G)], so[buf]).wait()

    for s in range(NG + LAG):
        if s < NG:
            buf = s % NBUF
            if s >= NBUF:
                _out_wait(buf)
            _gather(s, buf)
        if s >= LAG:
            s2 = s - LAG
            buf2 = s2 % NBUF
            _gather_wait(s2, buf2)
            _out(s2, buf2)
    for s2 in range(NG - NBUF, NG):
        _out_wait(s2 % NBUF)


_sc_kernel = functools.partial(
    pl.kernel,
    out_type=jax.ShapeDtypeStruct((NUM_TOKENS, DIM), jnp.float32),
    mesh=plsc.VectorSubcoreMesh(core_axis_name="c", subcore_axis_name="s"),
    compiler_params=pltpu.CompilerParams(needs_layout_passes=False),
    scratch_types=(
        [pltpu.VMEM((SEQ,), jnp.int32), pltpu.VMEM((NG, G), jnp.int32)]
        + [pltpu.VMEM((G, DIM), jnp.float32) for _ in range(NBUF)]
        + [pltpu.SemaphoreType.DMA for _ in range(2 * NBUF)]
    ),
)(_sc_body)


@jax.jit
def kernel(pad_mask):
    bsz, seq_len = pad_mask.shape
    mask = jnp.logical_not(pad_mask).astype(jnp.int32).reshape(-1)
    table = jnp.asarray(_TABLE)
    out = _sc_kernel(table, mask)
    return out.reshape(bsz, seq_len, DIM)


# speedup vs baseline: 2.2533x; 2.2533x over previous
import math
import functools

import jax
import jax.numpy as jnp
import numpy as np
from jax import lax
from jax.experimental import pallas as pl
from jax.experimental.pallas import tpu as pltpu
from jax.experimental.pallas import tpu_sc as plsc

BSZ = 4
SEQ = 8192
DIM = 1024
NUM_TOKENS = BSZ * SEQ
NW = 32
CHUNK = NUM_TOKENS // NW
CPR = SEQ // CHUNK
T = 16
NSLAB = CHUNK // T
LANES = 16
VPR = DIM // LANES
TABLE_ROWS = SEQ + 1 + T

_HALF = DIM // 2
_EMB_SCALE = math.log(10000.0) / (_HALF - 1)


def _build_table():
    freqs = np.exp(np.arange(_HALF, dtype=np.float32) * -_EMB_SCALE)
    ang = np.arange(SEQ + 1, dtype=np.float32)[:, None] * freqs[None, :]
    tab = np.concatenate([np.sin(ang), np.cos(ang)], axis=1).astype(np.float32)
    tab[0, :] = 0.0
    pad = np.zeros((TABLE_ROWS - tab.shape[0], DIM), np.float32)
    return np.concatenate([tab, pad], axis=0).reshape(-1)


_TABLE = _build_table()


def _sc_body(table_hbm, mask_hbm, out_hbm, row_v, src_sm, cnt_s,
             in0, in1, ob0, ob1, si0, si1, so0, so1):
    ins = (in0, in1)
    obs = (ob0, ob1)
    sis = (si0, si1)
    sos = (so0, so1)

    wid = lax.axis_index("s") * 2 + lax.axis_index("c")
    b = wid // CPR
    c = wid % CPR
    row_base = b * SEQ
    cbase = c * CHUNK
    out_base = row_base + cbase

    pltpu.sync_copy(mask_hbm.at[pl.ds(row_base, SEQ)], row_v)

    def _ofs(i, acc):
        return acc + jnp.sum(row_v[pl.ds(i * LANES, LANES)])

    offset = lax.fori_loop(0, c * (CHUNK // LANES), _ofs, jnp.int32(0))

    zvec = jnp.zeros((LANES,), jnp.float32)
    carry = offset
    for i in range(NSLAB):
        cnt_s[i] = carry
        v = row_v[pl.ds(cbase + i * LANES, LANES)]
        cum = plsc.cumsum(v)
        carry = carry + jnp.sum(v)
        src = jnp.where(v == 0, jnp.int32(T), cum - 1)
        for l in range(LANES):
            src_sm[i * LANES + l] = src[l]
        if i < 2 * VPR:
            ins[i // VPR][pl.ds(T * DIM + (i % VPR) * LANES, LANES)] = zvec

    def _in(s, p):
        pltpu.async_copy(
            table_hbm.at[pl.ds((cnt_s[s] + 1) * DIM, T * DIM)],
            ins[p].at[pl.ds(0, T * DIM)], sis[p])

    def _in_wait(s, p):
        pltpu.make_async_copy(
            table_hbm.at[pl.ds((cnt_s[s] + 1) * DIM, T * DIM)],
            ins[p].at[pl.ds(0, T * DIM)], sis[p]).wait()

    def _out(s, p):
        pltpu.async_copy(
            obs[p], out_hbm.at[pl.ds((out_base + s * T) * DIM, T * DIM)], sos[p])

    def _out_wait(s, p):
        pltpu.make_async_copy(
            obs[p], out_hbm.at[pl.ds(out_base * DIM, T * DIM)], sos[p]).wait()

    _in(0, 0)
    _in(1, 1)

    def _slab(it, _):
        for p in (0, 1):
            s = it * 2 + p
            _in_wait(s, p)

            @pl.when(s >= 2)
            def _():
                _out_wait(s - 2, p)

            ib, ob = ins[p], obs[p]

            def _tok(t, _c):
                src = src_sm[s * LANES + t]
                for j in range(VPR):
                    ob[pl.ds(t * DIM + j * LANES, LANES)] = (
                        ib[pl.ds(src * DIM + j * LANES, LANES)])
                return _c

            lax.fori_loop(0, T, _tok, 0)
            _out(s, p)

            @pl.when(s + 2 < NSLAB)
            def _():
                _in(s + 2, p)
        return 0

    lax.fori_loop(0, NSLAB // 2, _slab, 0)
    _out_wait(NSLAB - 2, 0)
    _out_wait(NSLAB - 1, 1)


_sc_kernel = functools.partial(
    pl.kernel,
    out_type=jax.ShapeDtypeStruct((NUM_TOKENS * DIM,), jnp.float32),
    mesh=plsc.VectorSubcoreMesh(core_axis_name="c", subcore_axis_name="s"),
    compiler_params=pltpu.CompilerParams(needs_layout_passes=False),
    scratch_types=[
        pltpu.VMEM((SEQ,), jnp.int32),
        pltpu.SMEM((CHUNK,), jnp.int32),
        pltpu.SMEM((NSLAB,), jnp.int32),
        pltpu.VMEM(((T + 1) * DIM,), jnp.float32),
        pltpu.VMEM(((T + 1) * DIM,), jnp.float32),
        pltpu.VMEM((T * DIM,), jnp.float32),
        pltpu.VMEM((T * DIM,), jnp.float32),
        pltpu.SemaphoreType.DMA,
        pltpu.SemaphoreType.DMA,
        pltpu.SemaphoreType.DMA,
        pltpu.SemaphoreType.DMA,
    ],
)(_sc_body)


@jax.jit
def kernel(pad_mask):
    bsz, seq_len = pad_mask.shape
    mask = jnp.logical_not(pad_mask).astype(jnp.int32).reshape(-1)
    table = jnp.asarray(_TABLE)
    out = _sc_kernel(table, mask)
    return out.reshape(bsz, seq_len, DIM)
